# Initial kernel scaffold; baseline (speedup 1.0000x reference)
import functools

import jax
import jax.numpy as jnp
from jax import lax
from jax.experimental import pallas as pl
from jax.experimental.pallas import tpu as pltpu
from jax.experimental.pallas import tpu_sc as plsc

NC, NS, L = 2, 16, 16
NW = NC * NS
C = 256
IG = 128

SP_D, TY_D, MV_D, IT_D = 64, 16, 32, 16
DENSE_W = 21
OUT_W = SP_D + 2 * TY_D + 4 * MV_D + IT_D + DENSE_W

SP_OFF = 0
TY_OFF = SP_D
MV_OFF = TY_OFF + 2 * TY_D
IT_OFF = MV_OFF + 4 * MV_D
DN_OFF = IT_OFF + IT_D


@functools.lru_cache(maxsize=None)
def _build(total_rows):
    rows_per_w = total_rows // NW
    nchunk = rows_per_w // C
    gpc = C // IG

    mesh = plsc.VectorSubcoreMesh(core_axis_name="c", subcore_axis_name="s")

    idx_t = pltpu.VMEM((gpc, IG), jnp.int32)
    scratch = [
        idx_t, idx_t, idx_t, idx_t, idx_t, idx_t, idx_t, idx_t,
        pltpu.VMEM((C, SP_D), jnp.float32),
        pltpu.VMEM((C, TY_D), jnp.float32),
        pltpu.VMEM((C, TY_D), jnp.float32),
        pltpu.VMEM((C, MV_D), jnp.float32),
        pltpu.VMEM((C, MV_D), jnp.float32),
        pltpu.VMEM((C, MV_D), jnp.float32),
        pltpu.VMEM((C, MV_D), jnp.float32),
        pltpu.VMEM((C, IT_D), jnp.float32),
        pltpu.VMEM((C, 1), jnp.float32),
        pltpu.VMEM((C, 7), jnp.float32),
        pltpu.VMEM((C, 7), jnp.float32),
        pltpu.VMEM((C, 4), jnp.float32),
        pltpu.VMEM((C, 1), jnp.float32),
        pltpu.VMEM((C, 1), jnp.float32),
        pltpu.VMEM((C, DENSE_W), jnp.float32),
        pltpu.SemaphoreType.DMA,
        pltpu.SemaphoreType.DMA,
        pltpu.SemaphoreType.DMA,
    ]

    @functools.partial(
        pl.kernel,
        out_type=jax.ShapeDtypeStruct((total_rows, OUT_W), jnp.float32),
        mesh=mesh,
        scratch_types=scratch,
    )
    def encoder(sp_i, t0_i, t1_i, m0_i, m1_i, m2_i, m3_i, it_i,
                hp, st, ss, pp, fa, ac,
                sp_t, ty_t, mv_t, it_t,
                out,
                sp_x, t0_x, t1_x, m0_x, m1_x, m2_x, m3_x, it_x,
                sp_r, t0_r, t1_r, m0_r, m1_r, m2_r, m3_r, it_r,
                hp_v, st_v, ss_v, pp_v, fa_v, ac_v, dn_v,
                sem_in, sem_g, sem_out):
        wid = lax.axis_index("s") * NC + lax.axis_index("c")
        lane = lax.iota(jnp.int32, L)

        idx_srcs = ((sp_i, sp_x), (t0_i, t0_x), (t1_i, t1_x),
                    (m0_i, m0_x), (m1_i, m1_x), (m2_i, m2_x), (m3_i, m3_x),
                    (it_i, it_x))
        dense_srcs = ((hp, hp_v), (st, st_v), (ss, ss_v),
                      (pp, pp_v), (fa, fa_v), (ac, ac_v))
        gathers = ((sp_t, sp_x, sp_r), (ty_t, t0_x, t0_r), (ty_t, t1_x, t1_r),
                   (mv_t, m0_x, m0_r), (mv_t, m1_x, m1_r),
                   (mv_t, m2_x, m2_r), (mv_t, m3_x, m3_r),
                   (it_t, it_x, it_r))
        outputs = ((sp_r, SP_OFF, SP_D),
                   (t0_r, TY_OFF, TY_D), (t1_r, TY_OFF + TY_D, TY_D),
                   (m0_r, MV_OFF, MV_D), (m1_r, MV_OFF + MV_D, MV_D),
                   (m2_r, MV_OFF + 2 * MV_D, MV_D), (m3_r, MV_OFF + 3 * MV_D, MV_D),
                   (it_r, IT_OFF, IT_D), (dn_v, DN_OFF, DENSE_W))
        dense_asm = ((hp_v, 1, 0), (st_v, 7, 1), (ss_v, 7, 8),
                     (pp_v, 4, 15), (fa_v, 1, 19), (ac_v, 1, 20))

        def chunk(ci, carry):
            rb = wid * rows_per_w + ci * C
            rbg = wid * (rows_per_w // IG) + ci * gpc

            ins = [pltpu.async_copy(h.at[pl.ds(rbg, gpc)], v, sem_in)
                   for h, v in idx_srcs]
            ins += [pltpu.async_copy(h.at[pl.ds(rb, C)], v, sem_in)
                    for h, v in dense_srcs]
            for d in ins:
                d.wait()

            gds = []
            for tab, xv, rv in gathers:
                for g in range(gpc):
                    gds.append(pltpu.async_copy(
                        tab.at[xv.at[g]], rv.at[pl.ds(g * IG, IG)], sem_g))

            def asm(i, c2):
                r16 = lane + i * L
                for src, w, off in dense_asm:
                    for j in range(w):
                        vals = plsc.load_gather(
                            src, [r16, jnp.full((L,), j, jnp.int32)])
                        plsc.store_scatter(
                            dn_v, [r16, jnp.full((L,), off + j, jnp.int32)],
                            vals)
                return c2
            lax.fori_loop(0, C // L, asm, 0)

            for d in gds:
                d.wait()

            outs = [pltpu.async_copy(buf, out.at[pl.ds(rb, C), pl.ds(off, w)],
                                     sem_out)
                    for buf, off, w in outputs]
            for d in outs:
                d.wait()
            return carry

        lax.fori_loop(0, nchunk, chunk, 0)

    return encoder


def kernel(species, types, moves, item, hp_pct, status, stat_stages, pp_pct,
           fainted, active, species_table, type_table, move_table, item_table):
    B, N = species.shape
    R = B * N
    G = R // IG

    def as_idx(a):
        return a.reshape(G, IG).astype(jnp.int32)

    out = _build(R)(
        as_idx(species),
        as_idx(types[..., 0]), as_idx(types[..., 1]),
        as_idx(moves[..., 0]), as_idx(moves[..., 1]),
        as_idx(moves[..., 2]), as_idx(moves[..., 3]),
        as_idx(item),
        hp_pct.reshape(R, 1), status.reshape(R, 7),
        stat_stages.reshape(R, 7), pp_pct.reshape(R, 4),
        fainted.reshape(R, 1), active.reshape(R, 1),
        species_table, type_table, move_table, item_table,
    )
    return out.reshape(B, N, OUT_W)


# speedup vs baseline: 2.1858x; 2.1858x over previous
import functools

import jax
import jax.numpy as jnp
from jax import lax
from jax.experimental import pallas as pl
from jax.experimental.pallas import tpu as pltpu
from jax.experimental.pallas import tpu_sc as plsc

NC, NS, L = 2, 16, 16
NW = NC * NS
C = 256
IG = 128

SP_D, TY_D, MV_D, IT_D = 64, 16, 32, 16
DENSE_W = 21
OUT_W = SP_D + 2 * TY_D + 4 * MV_D + IT_D + DENSE_W

SP_OFF = 0
TY_OFF = SP_D
MV_OFF = TY_OFF + 2 * TY_D
IT_OFF = MV_OFF + 4 * MV_D
DN_OFF = IT_OFF + IT_D


@functools.lru_cache(maxsize=None)
def _build(total_rows):
    rows_per_w = total_rows // NW
    nchunk = rows_per_w // C
    gpc = C // IG

    mesh = plsc.VectorSubcoreMesh(core_axis_name="c", subcore_axis_name="s")

    idx_t = pltpu.VMEM((gpc, IG), jnp.int32)
    scratch = [
        idx_t, idx_t, idx_t, idx_t, idx_t, idx_t, idx_t, idx_t,
        pltpu.VMEM((C, SP_D), jnp.float32),
        pltpu.VMEM((C, TY_D), jnp.float32),
        pltpu.VMEM((C, TY_D), jnp.float32),
        pltpu.VMEM((C, MV_D), jnp.float32),
        pltpu.VMEM((C, MV_D), jnp.float32),
        pltpu.VMEM((C, MV_D), jnp.float32),
        pltpu.VMEM((C, MV_D), jnp.float32),
        pltpu.VMEM((C, IT_D), jnp.float32),
        pltpu.VMEM((C, 1), jnp.float32),
        pltpu.VMEM((C, 7), jnp.float32),
        pltpu.VMEM((C, 7), jnp.float32),
        pltpu.VMEM((C, 4), jnp.float32),
        pltpu.VMEM((C, 1), jnp.float32),
        pltpu.VMEM((C, 1), jnp.float32),
        pltpu.VMEM((C, DENSE_W), jnp.float32),
        pltpu.SemaphoreType.DMA,
        pltpu.SemaphoreType.DMA,
        pltpu.SemaphoreType.DMA,
    ]

    @functools.partial(
        pl.kernel,
        out_type=jax.ShapeDtypeStruct((total_rows, OUT_W), jnp.float32),
        mesh=mesh,
        scratch_types=scratch,
        compiler_params=pltpu.CompilerParams(use_tc_tiling_on_sc=False,
                                             needs_layout_passes=False),
    )
    def encoder(sp_i, t0_i, t1_i, m0_i, m1_i, m2_i, m3_i, it_i,
                hp, st, ss, pp, fa, ac,
                sp_t, ty_t, mv_t, it_t,
                out,
                sp_x, t0_x, t1_x, m0_x, m1_x, m2_x, m3_x, it_x,
                sp_r, t0_r, t1_r, m0_r, m1_r, m2_r, m3_r, it_r,
                hp_v, st_v, ss_v, pp_v, fa_v, ac_v, dn_v,
                sem_in, sem_g, sem_out):
        wid = lax.axis_index("s") * NC + lax.axis_index("c")
        lane = lax.iota(jnp.int32, L)

        idx_srcs = ((sp_i, sp_x), (t0_i, t0_x), (t1_i, t1_x),
                    (m0_i, m0_x), (m1_i, m1_x), (m2_i, m2_x), (m3_i, m3_x),
                    (it_i, it_x))
        dense_srcs = ((hp, hp_v), (st, st_v), (ss, ss_v),
                      (pp, pp_v), (fa, fa_v), (ac, ac_v))
        gathers = ((sp_t, sp_x, sp_r), (ty_t, t0_x, t0_r), (ty_t, t1_x, t1_r),
                   (mv_t, m0_x, m0_r), (mv_t, m1_x, m1_r),
                   (mv_t, m2_x, m2_r), (mv_t, m3_x, m3_r),
                   (it_t, it_x, it_r))
        outputs = ((sp_r, SP_OFF, SP_D),
                   (t0_r, TY_OFF, TY_D), (t1_r, TY_OFF + TY_D, TY_D),
                   (m0_r, MV_OFF, MV_D), (m1_r, MV_OFF + MV_D, MV_D),
                   (m2_r, MV_OFF + 2 * MV_D, MV_D), (m3_r, MV_OFF + 3 * MV_D, MV_D),
                   (it_r, IT_OFF, IT_D), (dn_v, DN_OFF, DENSE_W))
        dense_asm = ((hp_v, 1, 0), (st_v, 7, 1), (ss_v, 7, 8),
                     (pp_v, 4, 15), (fa_v, 1, 19), (ac_v, 1, 20))

        def chunk(ci, carry):
            rb = wid * rows_per_w + ci * C
            rbg = wid * (rows_per_w // IG) + ci * gpc

            ins = [pltpu.async_copy(h.at[pl.ds(rbg, gpc)], v, sem_in)
                   for h, v in idx_srcs]
            ins += [pltpu.async_copy(h.at[pl.ds(rb, C)], v, sem_in)
                    for h, v in dense_srcs]
            for d in ins:
                d.wait()

            gds = []
            for tab, xv, rv in gathers:
                for g in range(gpc):
                    gds.append(pltpu.async_copy(
                        tab.at[xv.at[g]], rv.at[pl.ds(g * IG, IG)], sem_g))

            def asm(i, c2):
                r16 = lane + i * L
                for src, w, off in dense_asm:
                    for j in range(w):
                        vals = plsc.load_gather(
                            src, [r16, jnp.full((L,), j, jnp.int32)])
                        plsc.store_scatter(
                            dn_v, [r16, jnp.full((L,), off + j, jnp.int32)],
                            vals)
                return c2
            lax.fori_loop(0, C // L, asm, 0)

            for d in gds:
                d.wait()

            outs = [pltpu.async_copy(buf, out.at[pl.ds(rb, C), pl.ds(off, w)],
                                     sem_out)
                    for buf, off, w in outputs]
            for d in outs:
                d.wait()
            return carry

        lax.fori_loop(0, nchunk, chunk, 0)

    return encoder


def kernel(species, types, moves, item, hp_pct, status, stat_stages, pp_pct,
           fainted, active, species_table, type_table, move_table, item_table):
    B, N = species.shape
    R = B * N
    G = R // IG

    def as_idx(a):
        return a.reshape(G, IG).astype(jnp.int32)

    out = _build(R)(
        as_idx(species),
        as_idx(types[..., 0]), as_idx(types[..., 1]),
        as_idx(moves[..., 0]), as_idx(moves[..., 1]),
        as_idx(moves[..., 2]), as_idx(moves[..., 3]),
        as_idx(item),
        hp_pct.reshape(R, 1), status.reshape(R, 7),
        stat_stages.reshape(R, 7), pp_pct.reshape(R, 4),
        fainted.reshape(R, 1), active.reshape(R, 1),
        species_table, type_table, move_table, item_table,
    )
    return out.reshape(B, N, OUT_W)


# speedup vs baseline: 2.1944x; 1.0039x over previous
import functools

import jax
import jax.numpy as jnp
from jax import lax
from jax.experimental import pallas as pl
from jax.experimental.pallas import tpu as pltpu
from jax.experimental.pallas import tpu_sc as plsc

NC, NS, L = 2, 16, 16
NW = NC * NS
C = 128

SP_D, TY_D, MV_D, IT_D = 64, 16, 32, 16
DENSE_W = 21
OUT_W = SP_D + 2 * TY_D + 4 * MV_D + IT_D + DENSE_W

SP_OFF = 0
TY_OFF = SP_D
MV_OFF = TY_OFF + 2 * TY_D
IT_OFF = MV_OFF + 4 * MV_D
DN_OFF = IT_OFF + IT_D

SEGS = ((SP_D, SP_OFF),
        (TY_D, TY_OFF), (TY_D, TY_OFF + TY_D),
        (MV_D, MV_OFF), (MV_D, MV_OFF + MV_D),
        (MV_D, MV_OFF + 2 * MV_D), (MV_D, MV_OFF + 3 * MV_D),
        (IT_D, IT_OFF))
DENSE_SEGS = ((1, DN_OFF + 0), (7, DN_OFF + 1), (7, DN_OFF + 8),
              (4, DN_OFF + 15), (1, DN_OFF + 19), (1, DN_OFF + 20))


@functools.lru_cache(maxsize=None)
def _build(total_rows):
    rows_per_w = total_rows // NW
    nchunk = rows_per_w // C
    nc2 = nchunk // 2

    mesh = plsc.VectorSubcoreMesh(core_axis_name="c", subcore_axis_name="s")

    def one_set():
        return ([pltpu.VMEM((1, C), jnp.int32) for _ in range(8)]
                + [pltpu.VMEM((C, w), jnp.float32) for w, _ in DENSE_SEGS]
                + [pltpu.VMEM((C, w), jnp.float32) for w, _ in SEGS]
                + [pltpu.VMEM((C, DENSE_W), jnp.float32),
                   pltpu.SemaphoreType.DMA,
                   pltpu.SemaphoreType.DMA,
                   pltpu.SemaphoreType.DMA])

    @functools.partial(
        pl.kernel,
        out_type=jax.ShapeDtypeStruct((total_rows, OUT_W), jnp.float32),
        mesh=mesh,
        scratch_types=one_set() + one_set(),
        compiler_params=pltpu.CompilerParams(use_tc_tiling_on_sc=False,
                                             needs_layout_passes=False),
    )
    def encoder(sp_i, t0_i, t1_i, m0_i, m1_i, m2_i, m3_i, it_i,
                hp, st, ss, pp, fa, ac,
                sp_t, ty_t, mv_t, it_t,
                out, *scr):
        wid = lax.axis_index("s") * NC + lax.axis_index("c")
        lane = lax.iota(jnp.int32, L)

        idx_hbm = (sp_i, t0_i, t1_i, m0_i, m1_i, m2_i, m3_i, it_i)
        dense_hbm = (hp, st, ss, pp, fa, ac)
        tables = (sp_t, ty_t, ty_t, mv_t, mv_t, mv_t, mv_t, it_t)

        def split_set(s):
            return s[0:8], s[8:14], s[14:22], s[22], s[23], s[24], s[25]
        nset = len(scr) // 2
        sets = (split_set(scr[:nset]), split_set(scr[nset:]))

        def fire_ins(c, bs):
            idx_v, dn_v, _, _, sem_in, _, _ = bs
            rb = wid * rows_per_w + c * C
            rbg = wid * nchunk + c
            for h, v in zip(idx_hbm, idx_v):
                pltpu.async_copy(h.at[pl.ds(rbg, 1)], v, sem_in)
            for h, v in zip(dense_hbm, dn_v):
                pltpu.async_copy(h.at[pl.ds(rb, C)], v, sem_in)

        def wait_ins(bs):
            idx_v, dn_v, _, _, sem_in, _, _ = bs
            for h, v in zip(idx_hbm, idx_v):
                pltpu.make_async_copy(h.at[pl.ds(0, 1)], v, sem_in).wait()
            for h, v in zip(dense_hbm, dn_v):
                pltpu.make_async_copy(h.at[pl.ds(0, C)], v, sem_in).wait()

        def wait_out(bs):
            _, _, rows_v, dstage, _, _, sem_out = bs
            for (w, off), rv in zip(SEGS, rows_v):
                pltpu.make_async_copy(
                    rv, out.at[pl.ds(0, C), pl.ds(off, w)], sem_out).wait()
            pltpu.make_async_copy(
                dstage, out.at[pl.ds(0, C), pl.ds(DN_OFF, DENSE_W)],
                sem_out).wait()

        def run_chunk(c, bs):
            idx_v, dn_v, rows_v, dstage, sem_in, sem_g, sem_out = bs
            rb = wid * rows_per_w + c * C
            gds = []
            for tab, xv, rv in zip(tables, idx_v, rows_v):
                gds.append(pltpu.async_copy(tab.at[xv.at[0]], rv, sem_g))

            def asm(i, c2):
                r16 = lane + i * L
                for src, (w, off) in zip(dn_v, DENSE_SEGS):
                    for j in range(w):
                        vals = plsc.load_gather(
                            src, [r16, jnp.full((L,), j, jnp.int32)])
                        plsc.store_scatter(
                            dstage,
                            [r16, jnp.full((L,), off - DN_OFF + j, jnp.int32)],
                            vals)
                return c2
            lax.fori_loop(0, C // L, asm, 0)

            for d in gds:
                d.wait()
            for (w, off), rv in zip(SEGS, rows_v):
                pltpu.async_copy(
                    rv, out.at[pl.ds(rb, C), pl.ds(off, w)], sem_out)
            pltpu.async_copy(
                dstage, out.at[pl.ds(rb, C), pl.ds(DN_OFF, DENSE_W)], sem_out)

        fire_ins(0, sets[0])

        def body(c2, carry):
            for half in range(2):
                c = 2 * c2 + half
                bs = sets[half]

                @pl.when(c2 > 0)
                def _():
                    wait_out(bs)
                wait_ins(bs)
                if half == 0:
                    fire_ins(c + 1, sets[1])
                else:
                    @pl.when(c2 < nc2 - 1)
                    def _():
                        fire_ins(c + 1, sets[0])
                run_chunk(c, bs)
            return carry

        lax.fori_loop(0, nc2, body, 0)
        wait_out(sets[0])
        wait_out(sets[1])

    return encoder


def kernel(species, types, moves, item, hp_pct, status, stat_stages, pp_pct,
           fainted, active, species_table, type_table, move_table, item_table):
    B, N = species.shape
    R = B * N
    G = R // C

    def as_idx(a):
        return a.reshape(G, C).astype(jnp.int32)

    out = _build(R)(
        as_idx(species),
        as_idx(types[..., 0]), as_idx(types[..., 1]),
        as_idx(moves[..., 0]), as_idx(moves[..., 1]),
        as_idx(moves[..., 2]), as_idx(moves[..., 3]),
        as_idx(item),
        hp_pct.reshape(R, 1), status.reshape(R, 7),
        stat_stages.reshape(R, 7), pp_pct.reshape(R, 4),
        fainted.reshape(R, 1), active.reshape(R, 1),
        species_table, type_table, move_table, item_table,
    )
    return out.reshape(B, N, OUT_W)


# speedup vs baseline: 3.1030x; 1.4140x over previous
import functools

import jax
import jax.numpy as jnp
from jax import lax
from jax.experimental import pallas as pl
from jax.experimental.pallas import tpu as pltpu
from jax.experimental.pallas import tpu_sc as plsc

NC, NS, L = 2, 16, 16
NW = NC * NS
CHK = 4096
UNROLL = 8

N_SLOT = 6
NB = 16384
SP_D, TY_D, MV_D, IT_D = 64, 16, 32, 16
DENSE_W = 21
OUT_W = SP_D + 2 * TY_D + 4 * MV_D + IT_D + DENSE_W
SP_OFF, TY_OFF, MV_OFF = 0, 64, 96
IT_OFF, DN_OFF = 224, 240

DENSE_SEGS = ((1, DN_OFF + 0), (7, DN_OFF + 1), (7, DN_OFF + 8),
              (4, DN_OFF + 15), (1, DN_OFF + 19), (1, DN_OFF + 20))

NCHUNK = N_SLOT * (NB // CHK)


@functools.lru_cache(maxsize=None)
def _build():
    mesh = plsc.VectorSubcoreMesh(core_axis_name="c", subcore_axis_name="s")

    scratch = [
        pltpu.VMEM((100000,), jnp.float32),
        pltpu.VMEM((CHK,), jnp.int32),
        pltpu.VMEM((CHK,), jnp.int32),
        pltpu.VMEM((CHK,), jnp.float32),
        pltpu.VMEM((CHK,), jnp.float32),
        pltpu.SemaphoreType.DMA,
        pltpu.SemaphoreType.DMA,
        pltpu.SemaphoreType.DMA,
        pltpu.SemaphoreType.DMA,
    ]

    @functools.partial(
        pl.kernel,
        out_type=jax.ShapeDtypeStruct((N_SLOT, OUT_W, NB), jnp.float32),
        mesh=mesh,
        scratch_types=scratch,
        compiler_params=pltpu.CompilerParams(use_tc_tiling_on_sc=False,
                                             needs_layout_passes=False),
    )
    def encoder(sp_i, ty_i, mv_i, it_i,
                hp, st, ss, pp, fa, ac,
                sp_t, ty_t, mv_t, it_t,
                out,
                plane, idx0, idx1, ob0, ob1,
                sem_i0, sem_i1, sem_o0, sem_o1):
        wid = lax.axis_index("s") * NC + lax.axis_index("c")
        idx_b = (idx0, idx1)
        out_b = (ob0, ob1)
        sem_i = (sem_i0, sem_i1)
        sem_o = (sem_o0, sem_o1)
        dense_refs = (hp, st, ss, pp, fa, ac)

        p = 0
        for fi, (w, off) in enumerate(DENSE_SEGS):
            for n in range(N_SLOT):
                for j in range(w):
                    owner = p % NW
                    p += 1

                    @pl.when(wid == owner)
                    def _(fi=fi, n=n, j=j, off=off):
                        pltpu.sync_copy(dense_refs[fi].at[n, j],
                                        out.at[n, off + j])

        def task(idx_src, cg):
            def fire_idx(q, par):
                n, b0 = q // (NB // CHK), (q % (NB // CHK)) * CHK
                pltpu.async_copy(idx_src(n, b0), idx_b[par], sem_i[par])

            def wait_idx(par):
                pltpu.make_async_copy(idx_src(0, 0), idx_b[par],
                                      sem_i[par]).wait()

            def fire_out(q, par):
                n, b0 = q // (NB // CHK), (q % (NB // CHK)) * CHK
                pltpu.async_copy(out_b[par], out.at[n, cg, pl.ds(b0, CHK)],
                                 sem_o[par])

            def wait_out(par):
                pltpu.make_async_copy(out_b[par], out.at[0, 0, pl.ds(0, CHK)],
                                      sem_o[par]).wait()

            fire_idx(0, 0)
            fire_idx(1, 1)

            def body(k, carry):
                for par in range(2):
                    q = 2 * k + par
                    wait_idx(par)

                    @pl.when(k > 0)
                    def _(par=par):
                        wait_out(par)

                    def inner(i, c2, par=par):
                        base = i * (L * UNROLL)
                        for u in range(UNROLL):
                            o = base + u * L
                            iv = idx_b[par][pl.ds(o, L)]
                            out_b[par][pl.ds(o, L)] = plsc.load_gather(
                                plane, [iv])
                        return c2
                    lax.fori_loop(0, CHK // (L * UNROLL), inner, 0)

                    @pl.when(k < NCHUNK // 2 - 1)
                    def _(q=q, par=par):
                        fire_idx(q + 2, par)
                    fire_out(q, par)
                return carry

            lax.fori_loop(0, NCHUNK // 2, body, 0)
            wait_out(0)
            wait_out(1)

        for s in range(2):
            c = wid * 2 + s
            pltpu.sync_copy(sp_t.at[c], plane)
            task(lambda n, b0: sp_i.at[n, pl.ds(b0, CHK)], SP_OFF + c)

        pltpu.sync_copy(mv_t.at[wid], plane)
        for j in range(4):
            task(lambda n, b0, j=j: mv_i.at[n, j, pl.ds(b0, CHK)],
                 MV_OFF + j * MV_D + wid)

        @pl.when(wid % 2 == 1)
        def _():
            pltpu.sync_copy(ty_t.at[wid // 2], plane.at[pl.ds(0, 1000)])
            for j in range(2):
                task(lambda n, b0, j=j: ty_i.at[n, j, pl.ds(b0, CHK)],
                     TY_OFF + j * TY_D + wid // 2)

        @pl.when(wid % 2 == 0)
        def _():
            pltpu.sync_copy(it_t.at[wid // 2], plane.at[pl.ds(0, 1000)])
            task(lambda n, b0: it_i.at[n, pl.ds(b0, CHK)], IT_OFF + wid // 2)

    return encoder


def kernel(species, types, moves, item, hp_pct, status, stat_stages, pp_pct,
           fainted, active, species_table, type_table, move_table, item_table):
    out = _build()(
        jnp.transpose(species, (1, 0)),
        jnp.transpose(types, (1, 2, 0)),
        jnp.transpose(moves, (1, 2, 0)),
        jnp.transpose(item, (1, 0)),
        jnp.transpose(hp_pct, (1, 2, 0)),
        jnp.transpose(status, (1, 2, 0)),
        jnp.transpose(stat_stages, (1, 2, 0)),
        jnp.transpose(pp_pct, (1, 2, 0)),
        jnp.transpose(fainted, (1, 2, 0)),
        jnp.transpose(active, (1, 2, 0)),
        jnp.transpose(species_table, (1, 0)),
        jnp.transpose(type_table, (1, 0)),
        jnp.transpose(move_table, (1, 0)),
        jnp.transpose(item_table, (1, 0)),
    )
    return jnp.transpose(out, (2, 0, 1))


# speedup vs baseline: 4.4294x; 1.4275x over previous
import functools

import jax
import jax.numpy as jnp
from jax import lax
from jax.experimental import pallas as pl
from jax.experimental.pallas import tpu as pltpu
from jax.experimental.pallas import tpu_sc as plsc

NC, NS, L = 2, 16, 16
NW = NC * NS
CHK = 4096
UNROLL = 8

N_SLOT = 6
NB = 16384
SP_D, TY_D, MV_D, IT_D = 64, 16, 32, 16
DENSE_W = 21
OUT_W = SP_D + 2 * TY_D + 4 * MV_D + IT_D + DENSE_W
SP_OFF, TY_OFF, MV_OFF = 0, 64, 96
IT_OFF, DN_OFF = 224, 240

DENSE_SEGS = ((1, DN_OFF + 0), (7, DN_OFF + 1), (7, DN_OFF + 8),
              (4, DN_OFF + 15), (1, DN_OFF + 19), (1, DN_OFF + 20))

OUT_WP = 264
NCHUNK = N_SLOT * (NB // CHK)


@functools.lru_cache(maxsize=None)
def _build():
    mesh = plsc.VectorSubcoreMesh(core_axis_name="c", subcore_axis_name="s")

    scratch = [
        pltpu.VMEM((100000,), jnp.float32),
        pltpu.VMEM((CHK,), jnp.int32),
        pltpu.VMEM((CHK,), jnp.int32),
        pltpu.VMEM((CHK,), jnp.float32),
        pltpu.VMEM((CHK,), jnp.float32),
        pltpu.SemaphoreType.DMA,
        pltpu.SemaphoreType.DMA,
        pltpu.SemaphoreType.DMA,
        pltpu.SemaphoreType.DMA,
    ]

    @functools.partial(
        pl.kernel,
        out_type=jax.ShapeDtypeStruct((N_SLOT, OUT_WP, NB), jnp.float32),
        mesh=mesh,
        scratch_types=scratch,
        compiler_params=pltpu.CompilerParams(use_tc_tiling_on_sc=False,
                                             needs_layout_passes=False),
    )
    def encoder(sp_i, ty_i, mv_i, it_i,
                hp, st, ss, pp, fa, ac,
                sp_t, ty_t, mv_t, it_t,
                out,
                plane, idx0, idx1, ob0, ob1,
                sem_i0, sem_i1, sem_o0, sem_o1):
        wid = lax.axis_index("s") * NC + lax.axis_index("c")
        idx_b = (idx0, idx1)
        out_b = (ob0, ob1)
        sem_i = (sem_i0, sem_i1)
        sem_o = (sem_o0, sem_o1)
        dense_refs = (hp, st, ss, pp, fa, ac)

        p = 0
        for fi, (w, off) in enumerate(DENSE_SEGS):
            for n in range(N_SLOT):
                for j in range(w):
                    owner = p % NW
                    p += 1

                    @pl.when(wid == owner)
                    def _(fi=fi, n=n, j=j, off=off):
                        pltpu.sync_copy(dense_refs[fi].at[n, j],
                                        out.at[n, off + j])

        def task(idx_src, cg):
            def fire_idx(q, par):
                n, b0 = q // (NB // CHK), (q % (NB // CHK)) * CHK
                pltpu.async_copy(idx_src(n, b0), idx_b[par], sem_i[par])

            def wait_idx(par):
                pltpu.make_async_copy(idx_src(0, 0), idx_b[par],
                                      sem_i[par]).wait()

            def fire_out(q, par):
                n, b0 = q // (NB // CHK), (q % (NB // CHK)) * CHK
                pltpu.async_copy(out_b[par], out.at[n, cg, pl.ds(b0, CHK)],
                                 sem_o[par])

            def wait_out(par):
                pltpu.make_async_copy(out_b[par], out.at[0, 0, pl.ds(0, CHK)],
                                      sem_o[par]).wait()

            fire_idx(0, 0)
            fire_idx(1, 1)

            def body(k, carry):
                for par in range(2):
                    q = 2 * k + par
                    wait_idx(par)

                    @pl.when(k > 0)
                    def _(par=par):
                        wait_out(par)

                    def inner(i, c2, par=par):
                        base = i * (L * UNROLL)
                        for u in range(UNROLL):
                            o = base + u * L
                            iv = idx_b[par][pl.ds(o, L)]
                            out_b[par][pl.ds(o, L)] = plsc.load_gather(
                                plane, [iv])
                        return c2
                    lax.fori_loop(0, CHK // (L * UNROLL), inner, 0)

                    @pl.when(k < NCHUNK // 2 - 1)
                    def _(q=q, par=par):
                        fire_idx(q + 2, par)
                    fire_out(q, par)
                return carry

            lax.fori_loop(0, NCHUNK // 2, body, 0)
            wait_out(0)
            wait_out(1)

        for s in range(2):
            c = wid * 2 + s
            pltpu.sync_copy(sp_t.at[c], plane)
            task(lambda n, b0: sp_i.at[n, pl.ds(b0, CHK)], SP_OFF + c)

        pltpu.sync_copy(mv_t.at[wid], plane)
        for j in range(4):
            task(lambda n, b0, j=j: mv_i.at[n, j, pl.ds(b0, CHK)],
                 MV_OFF + j * MV_D + wid)

        @pl.when(wid % 2 == 1)
        def _():
            pltpu.sync_copy(ty_t.at[wid // 2], plane.at[pl.ds(0, 1000)])
            for j in range(2):
                task(lambda n, b0, j=j: ty_i.at[n, j, pl.ds(b0, CHK)],
                     TY_OFF + j * TY_D + wid // 2)

        @pl.when(wid % 2 == 0)
        def _():
            pltpu.sync_copy(it_t.at[wid // 2], plane.at[pl.ds(0, 1000)])
            task(lambda n, b0: it_i.at[n, pl.ds(b0, CHK)], IT_OFF + wid // 2)

    return encoder


def kernel(species, types, moves, item, hp_pct, status, stat_stages, pp_pct,
           fainted, active, species_table, type_table, move_table, item_table):
    out = _build()(
        jnp.transpose(species, (1, 0)),
        jnp.transpose(types, (1, 2, 0)),
        jnp.transpose(moves, (1, 2, 0)),
        jnp.transpose(item, (1, 0)),
        jnp.transpose(hp_pct, (1, 2, 0)),
        jnp.transpose(status, (1, 2, 0)),
        jnp.transpose(stat_stages, (1, 2, 0)),
        jnp.transpose(pp_pct, (1, 2, 0)),
        jnp.transpose(fainted, (1, 2, 0)),
        jnp.transpose(active, (1, 2, 0)),
        jnp.transpose(species_table, (1, 0)),
        jnp.transpose(type_table, (1, 0)),
        jnp.transpose(move_table, (1, 0)),
        jnp.transpose(item_table, (1, 0)),
    )
    return jnp.transpose(out[:, :OUT_W, :], (2, 0, 1))


# speedup vs baseline: 4.4301x; 1.0002x over previous
import functools

import jax
import jax.numpy as jnp
from jax import lax
from jax.experimental import pallas as pl
from jax.experimental.pallas import tpu as pltpu
from jax.experimental.pallas import tpu_sc as plsc

NC, NS, L = 2, 16, 16
NW = NC * NS
CHK = 4096
UNROLL = 16

N_SLOT = 6
NB = 16384
SP_D, TY_D, MV_D, IT_D = 64, 16, 32, 16
DENSE_W = 21
OUT_W = SP_D + 2 * TY_D + 4 * MV_D + IT_D + DENSE_W
SP_OFF, TY_OFF, MV_OFF = 0, 64, 96
IT_OFF, DN_OFF = 224, 240

DENSE_SEGS = ((1, DN_OFF + 0), (7, DN_OFF + 1), (7, DN_OFF + 8),
              (4, DN_OFF + 15), (1, DN_OFF + 19), (1, DN_OFF + 20))

OUT_WP = 264
NCHUNK = N_SLOT * (NB // CHK)


@functools.lru_cache(maxsize=None)
def _build():
    mesh = plsc.VectorSubcoreMesh(core_axis_name="c", subcore_axis_name="s")

    scratch = [
        pltpu.VMEM((100000,), jnp.float32),
        pltpu.VMEM((CHK,), jnp.int32),
        pltpu.VMEM((CHK,), jnp.int32),
        pltpu.VMEM((CHK,), jnp.float32),
        pltpu.VMEM((CHK,), jnp.float32),
        pltpu.SemaphoreType.DMA,
        pltpu.SemaphoreType.DMA,
        pltpu.SemaphoreType.DMA,
        pltpu.SemaphoreType.DMA,
    ]

    @functools.partial(
        pl.kernel,
        out_type=jax.ShapeDtypeStruct((N_SLOT, OUT_WP, NB), jnp.float32),
        mesh=mesh,
        scratch_types=scratch,
        compiler_params=pltpu.CompilerParams(use_tc_tiling_on_sc=False,
                                             needs_layout_passes=False),
    )
    def encoder(sp_i, ty_i, mv_i, it_i,
                hp, st, ss, pp, fa, ac,
                sp_t, ty_t, mv_t, it_t,
                out,
                plane, idx0, idx1, ob0, ob1,
                sem_i0, sem_i1, sem_o0, sem_o1):
        wid = lax.axis_index("s") * NC + lax.axis_index("c")
        idx_b = (idx0, idx1)
        out_b = (ob0, ob1)
        sem_i = (sem_i0, sem_i1)
        sem_o = (sem_o0, sem_o1)
        dense_refs = (hp, st, ss, pp, fa, ac)

        p = 0
        for fi, (w, off) in enumerate(DENSE_SEGS):
            for n in range(N_SLOT):
                for j in range(w):
                    owner = p % NW
                    p += 1

                    @pl.when(wid == owner)
                    def _(fi=fi, n=n, j=j, off=off):
                        pltpu.sync_copy(dense_refs[fi].at[n, j],
                                        out.at[n, off + j])

        def task(idx_src, cg):
            def fire_idx(q, par):
                n, b0 = q // (NB // CHK), (q % (NB // CHK)) * CHK
                pltpu.async_copy(idx_src(n, b0), idx_b[par], sem_i[par])

            def wait_idx(par):
                pltpu.make_async_copy(idx_src(0, 0), idx_b[par],
                                      sem_i[par]).wait()

            def fire_out(q, par):
                n, b0 = q // (NB // CHK), (q % (NB // CHK)) * CHK
                pltpu.async_copy(out_b[par], out.at[n, cg, pl.ds(b0, CHK)],
                                 sem_o[par])

            def wait_out(par):
                pltpu.make_async_copy(out_b[par], out.at[0, 0, pl.ds(0, CHK)],
                                      sem_o[par]).wait()

            fire_idx(0, 0)
            fire_idx(1, 1)

            def body(k, carry):
                for par in range(2):
                    q = 2 * k + par
                    wait_idx(par)

                    @pl.when(k > 0)
                    def _(par=par):
                        wait_out(par)

                    def inner(i, c2, par=par):
                        base = i * (L * UNROLL)
                        for u in range(UNROLL):
                            o = base + u * L
                            iv = idx_b[par][pl.ds(o, L)]
                            out_b[par][pl.ds(o, L)] = plsc.load_gather(
                                plane, [iv])
                        return c2
                    lax.fori_loop(0, CHK // (L * UNROLL), inner, 0)

                    @pl.when(k < NCHUNK // 2 - 1)
                    def _(q=q, par=par):
                        fire_idx(q + 2, par)
                    fire_out(q, par)
                return carry

            lax.fori_loop(0, NCHUNK // 2, body, 0)
            wait_out(0)
            wait_out(1)

        for s in range(2):
            c = wid * 2 + s
            pltpu.sync_copy(sp_t.at[c], plane)
            task(lambda n, b0: sp_i.at[n, pl.ds(b0, CHK)], SP_OFF + c)

        pltpu.sync_copy(mv_t.at[wid], plane)
        for j in range(4):
            task(lambda n, b0, j=j: mv_i.at[n, j, pl.ds(b0, CHK)],
                 MV_OFF + j * MV_D + wid)

        @pl.when(wid % 2 == 1)
        def _():
            pltpu.sync_copy(ty_t.at[wid // 2], plane.at[pl.ds(0, 1000)])
            for j in range(2):
                task(lambda n, b0, j=j: ty_i.at[n, j, pl.ds(b0, CHK)],
                     TY_OFF + j * TY_D + wid // 2)

        @pl.when(wid % 2 == 0)
        def _():
            pltpu.sync_copy(it_t.at[wid // 2], plane.at[pl.ds(0, 1000)])
            task(lambda n, b0: it_i.at[n, pl.ds(b0, CHK)], IT_OFF + wid // 2)

    return encoder


def kernel(species, types, moves, item, hp_pct, status, stat_stages, pp_pct,
           fainted, active, species_table, type_table, move_table, item_table):
    out = _build()(
        jnp.transpose(species, (1, 0)),
        jnp.transpose(types, (1, 2, 0)),
        jnp.transpose(moves, (1, 2, 0)),
        jnp.transpose(item, (1, 0)),
        jnp.transpose(hp_pct, (1, 2, 0)),
        jnp.transpose(status, (1, 2, 0)),
        jnp.transpose(stat_stages, (1, 2, 0)),
        jnp.transpose(pp_pct, (1, 2, 0)),
        jnp.transpose(fainted, (1, 2, 0)),
        jnp.transpose(active, (1, 2, 0)),
        jnp.transpose(species_table, (1, 0)),
        jnp.transpose(type_table, (1, 0)),
        jnp.transpose(move_table, (1, 0)),
        jnp.transpose(item_table, (1, 0)),
    )
    return jnp.transpose(out[:, :OUT_W, :], (2, 0, 1))


# speedup vs baseline: 4.9309x; 1.1130x over previous
import functools

import jax
import jax.numpy as jnp
from jax import lax
from jax.experimental import pallas as pl
from jax.experimental.pallas import tpu as pltpu
from jax.experimental.pallas import tpu_sc as plsc

NC, NS, L = 2, 16, 16
NW = NC * NS
CHK = 4096
UNROLL = 16

N_SLOT = 6
NB = 16384
SP_D, TY_D, MV_D, IT_D = 64, 16, 32, 16
DENSE_W = 21
OUT_W = SP_D + 2 * TY_D + 4 * MV_D + IT_D + DENSE_W
SP_OFF, TY_OFF, MV_OFF = 0, 64, 96
IT_OFF, DN_OFF = 224, 240

DENSE_SEGS = ((1, DN_OFF + 0), (7, DN_OFF + 1), (7, DN_OFF + 8),
              (4, DN_OFF + 15), (1, DN_OFF + 19), (1, DN_OFF + 20))

OUT_WP = 264
NCHUNK = N_SLOT * (NB // CHK)


@functools.lru_cache(maxsize=None)
def _build():
    mesh = plsc.VectorSubcoreMesh(core_axis_name="c", subcore_axis_name="s")

    scratch = [
        pltpu.VMEM((100000,), jnp.float32),
        pltpu.VMEM((CHK,), jnp.int32),
        pltpu.VMEM((CHK,), jnp.int32),
        pltpu.VMEM((CHK,), jnp.float32),
        pltpu.VMEM((CHK,), jnp.float32),
        pltpu.SemaphoreType.DMA,
        pltpu.SemaphoreType.DMA,
        pltpu.SemaphoreType.DMA,
        pltpu.SemaphoreType.DMA,
    ]

    @functools.partial(
        pl.kernel,
        out_type=jax.ShapeDtypeStruct((N_SLOT, OUT_WP, NB), jnp.float32),
        mesh=mesh,
        scratch_types=scratch,
        compiler_params=pltpu.CompilerParams(use_tc_tiling_on_sc=False,
                                             needs_layout_passes=False),
    )
    def encoder(sp_i, ty_i, mv_i, it_i,
                hp, st, ss, pp, fa, ac,
                sp_t, ty_t, mv_t, it_t,
                out,
                plane, idx0, idx1, ob0, ob1,
                sem_i0, sem_i1, sem_o0, sem_o1):
        wid = lax.axis_index("s") * NC + lax.axis_index("c")
        idx_b = (idx0, idx1)
        out_b = (ob0, ob1)
        sem_i = (sem_i0, sem_i1)
        sem_o = (sem_o0, sem_o1)
        dense_refs = (hp, st, ss, pp, fa, ac)

        p = 0
        for fi, (w, off) in enumerate(DENSE_SEGS):
            for n in range(N_SLOT):
                for j in range(w):
                    owner = p % NW
                    p += 1

                    @pl.when(wid == owner)
                    def _(fi=fi, n=n, j=j, off=off):
                        pltpu.sync_copy(dense_refs[fi].at[n, j],
                                        out.at[n, off + j])

        def task(idx_src, cg):
            def fire_idx(q, par):
                n, b0 = q // (NB // CHK), (q % (NB // CHK)) * CHK
                pltpu.async_copy(idx_src(n, b0), idx_b[par], sem_i[par])

            def wait_idx(par):
                pltpu.make_async_copy(idx_src(0, 0), idx_b[par],
                                      sem_i[par]).wait()

            def fire_out(q, par):
                n, b0 = q // (NB // CHK), (q % (NB // CHK)) * CHK
                pltpu.async_copy(out_b[par], out.at[n, cg, pl.ds(b0, CHK)],
                                 sem_o[par])

            def wait_out(par):
                pltpu.make_async_copy(out_b[par], out.at[0, 0, pl.ds(0, CHK)],
                                      sem_o[par]).wait()

            fire_idx(0, 0)
            fire_idx(1, 1)

            def body(k, carry):
                for par in range(2):
                    q = 2 * k + par
                    wait_idx(par)

                    @pl.when(k > 0)
                    def _(par=par):
                        wait_out(par)

                    @plsc.parallel_loop(0, CHK, L * UNROLL)
                    def inner(base, par=par):
                        for u in range(UNROLL):
                            o = base + u * L
                            iv = idx_b[par][pl.ds(o, L)]
                            out_b[par][pl.ds(o, L)] = plsc.load_gather(
                                plane, [iv])

                    @pl.when(k < NCHUNK // 2 - 1)
                    def _(q=q, par=par):
                        fire_idx(q + 2, par)
                    fire_out(q, par)
                return carry

            lax.fori_loop(0, NCHUNK // 2, body, 0)
            wait_out(0)
            wait_out(1)

        for s in range(2):
            c = wid * 2 + s
            pltpu.sync_copy(sp_t.at[c], plane)
            task(lambda n, b0: sp_i.at[n, pl.ds(b0, CHK)], SP_OFF + c)

        pltpu.sync_copy(mv_t.at[wid], plane)
        for j in range(4):
            task(lambda n, b0, j=j: mv_i.at[n, j, pl.ds(b0, CHK)],
                 MV_OFF + j * MV_D + wid)

        @pl.when(wid % 2 == 1)
        def _():
            pltpu.sync_copy(ty_t.at[wid // 2], plane.at[pl.ds(0, 1000)])
            for j in range(2):
                task(lambda n, b0, j=j: ty_i.at[n, j, pl.ds(b0, CHK)],
                     TY_OFF + j * TY_D + wid // 2)

        @pl.when(wid % 2 == 0)
        def _():
            pltpu.sync_copy(it_t.at[wid // 2], plane.at[pl.ds(0, 1000)])
            task(lambda n, b0: it_i.at[n, pl.ds(b0, CHK)], IT_OFF + wid // 2)

    return encoder


def kernel(species, types, moves, item, hp_pct, status, stat_stages, pp_pct,
           fainted, active, species_table, type_table, move_table, item_table):
    out = _build()(
        jnp.transpose(species, (1, 0)),
        jnp.transpose(types, (1, 2, 0)),
        jnp.transpose(moves, (1, 2, 0)),
        jnp.transpose(item, (1, 0)),
        jnp.transpose(hp_pct, (1, 2, 0)),
        jnp.transpose(status, (1, 2, 0)),
        jnp.transpose(stat_stages, (1, 2, 0)),
        jnp.transpose(pp_pct, (1, 2, 0)),
        jnp.transpose(fainted, (1, 2, 0)),
        jnp.transpose(active, (1, 2, 0)),
        jnp.transpose(species_table, (1, 0)),
        jnp.transpose(type_table, (1, 0)),
        jnp.transpose(move_table, (1, 0)),
        jnp.transpose(item_table, (1, 0)),
    )
    return jnp.transpose(out[:, :OUT_W, :], (2, 0, 1))


# speedup vs baseline: 5.0671x; 1.0276x over previous
import functools

import jax
import jax.numpy as jnp
from jax import lax
from jax.experimental import pallas as pl
from jax.experimental.pallas import tpu as pltpu
from jax.experimental.pallas import tpu_sc as plsc

NC, NS, L = 2, 16, 16
NW = NC * NS
CHK = 4096
UNROLL = 16

N_SLOT = 6
NB = 16384
SP_D, TY_D, MV_D, IT_D = 64, 16, 32, 16
DENSE_W = 21
OUT_W = SP_D + 2 * TY_D + 4 * MV_D + IT_D + DENSE_W
SP_OFF, TY_OFF, MV_OFF = 0, 64, 96
IT_OFF, DN_OFF = 224, 240

DENSE_SEGS = ((1, DN_OFF + 0), (7, DN_OFF + 1), (7, DN_OFF + 8),
              (4, DN_OFF + 15), (1, DN_OFF + 19), (1, DN_OFF + 20))

OUT_WP = 264
NCHUNK = N_SLOT * (NB // CHK)


@functools.lru_cache(maxsize=None)
def _build():
    mesh = plsc.VectorSubcoreMesh(core_axis_name="c", subcore_axis_name="s")

    scratch = [
        pltpu.VMEM((100000,), jnp.float32),
        pltpu.VMEM((CHK,), jnp.int32),
        pltpu.VMEM((CHK,), jnp.int32),
        pltpu.VMEM((CHK,), jnp.float32),
        pltpu.VMEM((CHK,), jnp.float32),
        pltpu.SemaphoreType.DMA,
        pltpu.SemaphoreType.DMA,
        pltpu.SemaphoreType.DMA,
        pltpu.SemaphoreType.DMA,
    ]

    @functools.partial(
        pl.kernel,
        out_type=jax.ShapeDtypeStruct((N_SLOT, OUT_WP, NB), jnp.float32),
        mesh=mesh,
        scratch_types=scratch,
        compiler_params=pltpu.CompilerParams(use_tc_tiling_on_sc=False,
                                             needs_layout_passes=False),
    )
    def encoder(sp_i, ty_i, mv_i, it_i,
                hp, st, ss, pp, fa, ac,
                sp_t, ty_t, mv_t, it_t,
                out,
                plane, idx0, idx1, ob0, ob1,
                sem_i0, sem_i1, sem_o0, sem_o1):
        wid = lax.axis_index("s") * NC + lax.axis_index("c")
        idx_b = (idx0, idx1)
        out_b = (ob0, ob1)
        sem_i = (sem_i0, sem_i1)
        sem_o = (sem_o0, sem_o1)
        dense_refs = (hp, st, ss, pp, fa, ac)

        p = 0
        for fi, (w, off) in enumerate(DENSE_SEGS):
            for n in range(N_SLOT):
                for j in range(w):
                    owner = p % NW
                    p += 1

                    @pl.when(wid == owner)
                    def _(fi=fi, n=n, j=j, off=off):
                        pltpu.sync_copy(dense_refs[fi].at[n, j],
                                        out.at[n, off + j])

        def task(idx_src, cg):
            def fire_idx(q, par):
                n, b0 = q // (NB // CHK), (q % (NB // CHK)) * CHK
                pltpu.async_copy(idx_src(n, b0), idx_b[par], sem_i[par])

            def wait_idx(par):
                pltpu.make_async_copy(idx_src(0, 0), idx_b[par],
                                      sem_i[par]).wait()

            def fire_out(q, par):
                n, b0 = q // (NB // CHK), (q % (NB // CHK)) * CHK
                pltpu.async_copy(out_b[par], out.at[n, cg, pl.ds(b0, CHK)],
                                 sem_o[par])

            def wait_out(par):
                pltpu.make_async_copy(out_b[par], out.at[0, 0, pl.ds(0, CHK)],
                                      sem_o[par]).wait()

            fire_idx(0, 0)
            fire_idx(1, 1)

            def body(k, carry):
                for par in range(2):
                    q = 2 * k + par
                    wait_idx(par)

                    @pl.when(k > 0)
                    def _(par=par):
                        wait_out(par)

                    @plsc.parallel_loop(0, CHK, L, unroll=UNROLL)
                    def inner(o, par=par):
                        iv = idx_b[par][pl.ds(o, L)]
                        out_b[par][pl.ds(o, L)] = plsc.load_gather(plane, [iv])

                    @pl.when(k < NCHUNK // 2 - 1)
                    def _(q=q, par=par):
                        fire_idx(q + 2, par)
                    fire_out(q, par)
                return carry

            lax.fori_loop(0, NCHUNK // 2, body, 0)
            wait_out(0)
            wait_out(1)

        for s in range(2):
            c = wid * 2 + s
            pltpu.sync_copy(sp_t.at[c], plane)
            task(lambda n, b0: sp_i.at[n, pl.ds(b0, CHK)], SP_OFF + c)

        pltpu.sync_copy(mv_t.at[wid], plane)
        for j in range(4):
            task(lambda n, b0, j=j: mv_i.at[n, j, pl.ds(b0, CHK)],
                 MV_OFF + j * MV_D + wid)

        @pl.when(wid % 2 == 1)
        def _():
            pltpu.sync_copy(ty_t.at[wid // 2], plane.at[pl.ds(0, 1000)])
            for j in range(2):
                task(lambda n, b0, j=j: ty_i.at[n, j, pl.ds(b0, CHK)],
                     TY_OFF + j * TY_D + wid // 2)

        @pl.when(wid % 2 == 0)
        def _():
            pltpu.sync_copy(it_t.at[wid // 2], plane.at[pl.ds(0, 1000)])
            task(lambda n, b0: it_i.at[n, pl.ds(b0, CHK)], IT_OFF + wid // 2)

    return encoder


def kernel(species, types, moves, item, hp_pct, status, stat_stages, pp_pct,
           fainted, active, species_table, type_table, move_table, item_table):
    out = _build()(
        jnp.transpose(species, (1, 0)),
        jnp.transpose(types, (1, 2, 0)),
        jnp.transpose(moves, (1, 2, 0)),
        jnp.transpose(item, (1, 0)),
        jnp.transpose(hp_pct, (1, 2, 0)),
        jnp.transpose(status, (1, 2, 0)),
        jnp.transpose(stat_stages, (1, 2, 0)),
        jnp.transpose(pp_pct, (1, 2, 0)),
        jnp.transpose(fainted, (1, 2, 0)),
        jnp.transpose(active, (1, 2, 0)),
        jnp.transpose(species_table, (1, 0)),
        jnp.transpose(type_table, (1, 0)),
        jnp.transpose(move_table, (1, 0)),
        jnp.transpose(item_table, (1, 0)),
    )
    return jnp.transpose(out[:, :OUT_W, :], (2, 0, 1))


# speedup vs baseline: 5.0939x; 1.0053x over previous
import functools

import jax
import jax.numpy as jnp
from jax import lax
from jax.experimental import pallas as pl
from jax.experimental.pallas import tpu as pltpu
from jax.experimental.pallas import tpu_sc as plsc

NC, NS, L = 2, 16, 16
NW = NC * NS
CHK = 4096
UNROLL = 16

N_SLOT = 6
NB = 16384
SP_D, TY_D, MV_D, IT_D = 64, 16, 32, 16
DENSE_W = 21
OUT_W = SP_D + 2 * TY_D + 4 * MV_D + IT_D + DENSE_W
SP_OFF, TY_OFF, MV_OFF = 0, 64, 96
IT_OFF, DN_OFF = 224, 240

DENSE_SEGS = ((1, DN_OFF + 0), (7, DN_OFF + 1), (7, DN_OFF + 8),
              (4, DN_OFF + 15), (1, DN_OFF + 19), (1, DN_OFF + 20))

OUT_WP = 264
NCHUNK = N_SLOT * (NB // CHK)


@functools.lru_cache(maxsize=None)
def _build():
    mesh = plsc.VectorSubcoreMesh(core_axis_name="c", subcore_axis_name="s")

    scratch = [
        pltpu.VMEM((100000,), jnp.float32),
        pltpu.VMEM((CHK,), jnp.int32),
        pltpu.VMEM((CHK,), jnp.int32),
        pltpu.VMEM((CHK,), jnp.float32),
        pltpu.VMEM((CHK,), jnp.float32),
        pltpu.SemaphoreType.DMA,
        pltpu.SemaphoreType.DMA,
        pltpu.SemaphoreType.DMA,
        pltpu.SemaphoreType.DMA,
    ]

    @functools.partial(
        pl.kernel,
        out_type=jax.ShapeDtypeStruct((N_SLOT, OUT_WP, NB), jnp.float32),
        mesh=mesh,
        scratch_types=scratch,
        compiler_params=pltpu.CompilerParams(use_tc_tiling_on_sc=False,
                                             needs_layout_passes=False),
    )
    def encoder(sp_i, ty_i, mv_i, it_i,
                hp, st, ss, pp, fa, ac,
                sp_t, ty_t, mv_t, it_t,
                out,
                plane, idx0, idx1, ob0, ob1,
                sem_i0, sem_i1, sem_o0, sem_o1):
        wid = lax.axis_index("s") * NC + lax.axis_index("c")
        idx_b = (idx0, idx1)
        out_b = (ob0, ob1)
        sem_i = (sem_i0, sem_i1)
        sem_o = (sem_o0, sem_o1)
        dense_refs = (hp, st, ss, pp, fa, ac)

        p = 0
        for fi, (w, off) in enumerate(DENSE_SEGS):
            for n in range(N_SLOT):
                for j in range(w):
                    owner = p % NW
                    p += 1

                    @pl.when(wid == owner)
                    def _(fi=fi, n=n, j=j, off=off):
                        pltpu.sync_copy(dense_refs[fi].at[n, j],
                                        out.at[n, off + j])

        def task(idx_src, cg, q0=0, nq=NCHUNK):
            def fire_idx(q, par):
                n, b0 = q // (NB // CHK), (q % (NB // CHK)) * CHK
                pltpu.async_copy(idx_src(n, b0), idx_b[par], sem_i[par])

            def wait_idx(par):
                pltpu.make_async_copy(idx_src(0, 0), idx_b[par],
                                      sem_i[par]).wait()

            def fire_out(q, par):
                n, b0 = q // (NB // CHK), (q % (NB // CHK)) * CHK
                pltpu.async_copy(out_b[par], out.at[n, cg, pl.ds(b0, CHK)],
                                 sem_o[par])

            def wait_out(par):
                pltpu.make_async_copy(out_b[par], out.at[0, 0, pl.ds(0, CHK)],
                                      sem_o[par]).wait()

            fire_idx(q0, 0)
            fire_idx(q0 + 1, 1)

            def body(k, carry):
                for par in range(2):
                    q = q0 + 2 * k + par
                    wait_idx(par)

                    @pl.when(k > 0)
                    def _(par=par):
                        wait_out(par)

                    @plsc.parallel_loop(0, CHK, L, unroll=UNROLL)
                    def inner(o, par=par):
                        iv = idx_b[par][pl.ds(o, L)]
                        out_b[par][pl.ds(o, L)] = plsc.load_gather(plane, [iv])

                    @pl.when(k < nq // 2 - 1)
                    def _(q=q, par=par):
                        fire_idx(q + 2, par)
                    fire_out(q, par)
                return carry

            lax.fori_loop(0, nq // 2, body, 0)
            wait_out(0)
            wait_out(1)

        for s in range(2):
            c = wid * 2 + s
            pltpu.sync_copy(sp_t.at[c], plane)
            task(lambda n, b0: sp_i.at[n, pl.ds(b0, CHK)], SP_OFF + c)

        pltpu.sync_copy(mv_t.at[wid], plane)
        for j in range(4):
            task(lambda n, b0, j=j: mv_i.at[n, j, pl.ds(b0, CHK)],
                 MV_OFF + j * MV_D + wid)

        @pl.when(wid % 2 == 1)
        def _():
            pltpu.sync_copy(ty_t.at[wid // 2], plane.at[pl.ds(0, 1000)])
            task(lambda n, b0: ty_i.at[n, 0, pl.ds(b0, CHK)],
                 TY_OFF + wid // 2)
            task(lambda n, b0: ty_i.at[n, 1, pl.ds(b0, CHK)],
                 TY_OFF + TY_D + wid // 2, 0, NCHUNK // 2)

        @pl.when(wid % 2 == 0)
        def _():
            pltpu.sync_copy(it_t.at[wid // 2], plane.at[pl.ds(0, 1000)])
            task(lambda n, b0: it_i.at[n, pl.ds(b0, CHK)], IT_OFF + wid // 2)
            pltpu.sync_copy(ty_t.at[wid // 2], plane.at[pl.ds(0, 1000)])
            task(lambda n, b0: ty_i.at[n, 1, pl.ds(b0, CHK)],
                 TY_OFF + TY_D + wid // 2, NCHUNK // 2, NCHUNK // 2)

    return encoder


def kernel(species, types, moves, item, hp_pct, status, stat_stages, pp_pct,
           fainted, active, species_table, type_table, move_table, item_table):
    out = _build()(
        jnp.transpose(species, (1, 0)),
        jnp.transpose(types, (1, 2, 0)),
        jnp.transpose(moves, (1, 2, 0)),
        jnp.transpose(item, (1, 0)),
        jnp.transpose(hp_pct, (1, 2, 0)),
        jnp.transpose(status, (1, 2, 0)),
        jnp.transpose(stat_stages, (1, 2, 0)),
        jnp.transpose(pp_pct, (1, 2, 0)),
        jnp.transpose(fainted, (1, 2, 0)),
        jnp.transpose(active, (1, 2, 0)),
        jnp.transpose(species_table, (1, 0)),
        jnp.transpose(type_table, (1, 0)),
        jnp.transpose(move_table, (1, 0)),
        jnp.transpose(item_table, (1, 0)),
    )
    return jnp.transpose(out[:, :OUT_W, :], (2, 0, 1))
